# baseline (speedup 1.0000x reference)
import functools

import jax
import jax.numpy as jnp
from jax import lax
from jax.experimental import pallas as pl
from jax.experimental.pallas import tpu as pltpu
from jax.experimental.pallas import tpu_sc as plsc

R = 128
N = 32768
NC = 2
NS = 16
L = 16
NW = NC * NS
RPW = R // NW
BLK_V = 16
BLK_E = BLK_V * L
NBLK = N // BLK_E
FB_UNROLL = 4

_mesh = plsc.VectorSubcoreMesh(core_axis_name="c", subcore_axis_name="s")


def _lane_min(v):
    for s in (8, 4, 2, 1):
        perm = jnp.arange(L, dtype=jnp.int32) ^ s
        v = jnp.minimum(v, v.at[perm].get(mode="promise_in_bounds"))
    return v


def _argmin_one_row(buf, blkmin, iota):
    inf = jnp.float32(jnp.inf)
    inf_vec = jnp.full((L,), inf, jnp.float32)

    @plsc.parallel_loop(0, NBLK, carry=inf_vec, unroll=2)
    def gmin(b, g):
        e0 = b * BLK_E
        vs = [buf[pl.ds(e0 + k * L, L)] for k in range(BLK_V)]
        while len(vs) > 1:
            vs = [jnp.minimum(vs[i], vs[i + 1]) for i in range(0, len(vs), 2)]
        blkmin[pl.ds(b * L, L)] = vs[0]
        return jnp.minimum(g, vs[0])

    m = _lane_min(gmin)

    nb_vec = jnp.full((L,), NBLK, jnp.int32)

    @plsc.parallel_loop(0, NBLK, step=FB_UNROLL, carry=nb_vec, unroll=2)
    def bb(j, acc):
        for k in range(FB_UNROLL):
            jb = j + k
            bm = blkmin[pl.ds(jb * L, L)]
            acc = jnp.minimum(acc, jnp.where(bm == m, jb, NBLK))
        return acc

    bstar = _lane_min(bb)[0]

    big = jnp.int32(N)
    e0 = bstar * BLK_E
    cands = [jnp.full((L,), big, jnp.int32) for _ in range(4)]
    for k in range(BLK_V):
        v = buf[pl.ds(e0 + k * L, L)]
        pos = iota + (e0 + k * L)
        cands[k % 4] = jnp.minimum(cands[k % 4], jnp.where(v == m, pos, big))
    bi = jnp.minimum(jnp.minimum(cands[0], cands[1]),
                     jnp.minimum(cands[2], cands[3]))
    return _lane_min(bi)


@functools.partial(
    pl.kernel,
    mesh=_mesh,
    out_type=jax.ShapeDtypeStruct((NW, L), jnp.int32),
    scratch_types=[
        pltpu.VMEM((N,), jnp.float32),
        pltpu.VMEM((N,), jnp.float32),
        pltpu.VMEM((NBLK * L,), jnp.float32),
        pltpu.VMEM((L,), jnp.int32),
        pltpu.SemaphoreType.DMA,
        pltpu.SemaphoreType.DMA,
    ],
)
def _argmin_rows(t_hbm, out_hbm, buf0, buf1, blkmin, res_v, sem0, sem1):
    wid = lax.axis_index("s") * NC + lax.axis_index("c")
    base = wid * RPW
    bufs = (buf0, buf1)
    sems = (sem0, sem1)
    iota = lax.iota(jnp.int32, L)

    copies = {0: pltpu.async_copy(t_hbm.at[base], buf0, sem0)}
    res = jnp.zeros((L,), jnp.int32)
    for r in range(RPW):
        copies[r % 2].wait()
        if r + 1 < RPW:
            copies[(r + 1) % 2] = pltpu.async_copy(
                t_hbm.at[base + r + 1], bufs[(r + 1) % 2], sems[(r + 1) % 2])
        idx = _argmin_one_row(bufs[r % 2], blkmin, iota)
        res = jnp.where(iota == r, idx, res)
    res_v[...] = res
    pltpu.sync_copy(res_v, out_hbm.at[wid])


def kernel(tensor):
    out = _argmin_rows(tensor)
    return out[:, :RPW].reshape(R, 1)


# speedup vs baseline: 2.8162x; 2.8162x over previous
import functools

import jax
import jax.numpy as jnp
from jax import lax
from jax.experimental import pallas as pl
from jax.experimental.pallas import tpu as pltpu
from jax.experimental.pallas import tpu_sc as plsc

R = 128
N = 32768
NC = 2
NS = 16
L = 16
NW = NC * NS
RPW = R // NW
BLK_V = 16
BLK_E = BLK_V * L
NBLK = N // BLK_E
FB_UNROLL = 4

_mesh = plsc.VectorSubcoreMesh(core_axis_name="c", subcore_axis_name="s")


def _lane_min(v):
    for s in (8, 4, 2, 1):
        perm = jnp.arange(L, dtype=jnp.int32) ^ s
        v = jnp.minimum(v, v.at[perm].get(mode="promise_in_bounds"))
    return v


def _argmin_one_row(buf, blkmin, iota):
    inf = jnp.float32(jnp.inf)
    inf_vec = jnp.full((L,), inf, jnp.float32)

    @plsc.parallel_loop(0, NBLK, carry=inf_vec, unroll=2)
    def gmin(b, g):
        e0 = b * BLK_E
        vs = [buf[pl.ds(e0 + k * L, L)] for k in range(BLK_V)]
        while len(vs) > 1:
            vs = [jnp.minimum(vs[i], vs[i + 1]) for i in range(0, len(vs), 2)]
        blkmin[pl.ds(b * L, L)] = vs[0]
        return jnp.minimum(g, vs[0])

    m = _lane_min(gmin)

    nb_vec = jnp.full((L,), NBLK, jnp.int32)

    @plsc.parallel_loop(0, NBLK, step=FB_UNROLL, carry=nb_vec, unroll=2)
    def bb(j, acc):
        for k in range(FB_UNROLL):
            jb = j + k
            bm = blkmin[pl.ds(jb * L, L)]
            acc = jnp.minimum(acc, jnp.where(bm == m, jb, NBLK))
        return acc

    bstar = _lane_min(bb)[0]

    big = jnp.int32(N)
    e0 = bstar * BLK_E
    cands = [jnp.full((L,), big, jnp.int32) for _ in range(4)]
    for k in range(BLK_V):
        v = buf[pl.ds(e0 + k * L, L)]
        pos = iota + (e0 + k * L)
        cands[k % 4] = jnp.minimum(cands[k % 4], jnp.where(v == m, pos, big))
    bi = jnp.minimum(jnp.minimum(cands[0], cands[1]),
                     jnp.minimum(cands[2], cands[3]))
    return _lane_min(bi)


@functools.partial(
    pl.kernel,
    mesh=_mesh,
    out_type=jax.ShapeDtypeStruct((NW, L), jnp.int32),
    scratch_types=[
        pltpu.VMEM((N,), jnp.float32),
        pltpu.VMEM((N,), jnp.float32),
        pltpu.VMEM((NBLK * L,), jnp.float32),
        pltpu.VMEM((L,), jnp.int32),
        pltpu.SemaphoreType.DMA,
        pltpu.SemaphoreType.DMA,
    ],
)
def _argmin_rows(t_hbm, out_hbm, buf0, buf1, blkmin, res_v, sem0, sem1):
    wid = lax.axis_index("s") * NC + lax.axis_index("c")
    base = wid * RPW
    bufs = (buf0, buf1)
    sems = (sem0, sem1)
    iota = lax.iota(jnp.int32, L)

    copies = {0: pltpu.async_copy(t_hbm.at[base], buf0, sem0)}
    res = jnp.zeros((L,), jnp.int32)
    for r in range(RPW):
        copies[r % 2].wait()
        if r + 1 < RPW:
            copies[(r + 1) % 2] = pltpu.async_copy(
                t_hbm.at[base + r + 1], bufs[(r + 1) % 2], sems[(r + 1) % 2])
        idx = _argmin_one_row(bufs[r % 2], blkmin, iota)
        res = jnp.where(iota == r, idx, res)
    res_v[...] = res
    pltpu.sync_copy(res_v, out_hbm.at[wid])


TC_G = 8
TC_CB = N // TC_G


def _tc_argmin(x):
    rt = x.shape[0]

    def body(x_ref, o_ref, vacc, iacc):
        j = pl.program_id(0)
        av = jnp.where(j == 0, jnp.float32(jnp.inf), vacc[...])
        iv = jnp.where(j == 0, 0, iacc[...])
        lane = lax.broadcasted_iota(jnp.int32, (rt, 128), 1)
        for g in range(TC_CB // 128):
            xg = x_ref[:, pl.ds(g * 128, 128)]
            idxg = lane + (j * TC_CB + g * 128)
            mask = xg < av
            av = jnp.minimum(av, xg)
            iv = jnp.where(mask, idxg, iv)
        vacc[...] = av
        iacc[...] = iv

        @pl.when(j == TC_G - 1)
        def _():
            rv = jnp.min(av, axis=1, keepdims=True)
            ii = jnp.where(av == rv, iv, N)
            o_ref[...] = jnp.min(ii, axis=1, keepdims=True)

    return pl.pallas_call(
        body,
        grid=(TC_G,),
        in_specs=[pl.BlockSpec((rt, TC_CB), lambda j: (0, j))],
        out_specs=pl.BlockSpec((rt, 1), lambda j: (0, 0)),
        out_shape=jax.ShapeDtypeStruct((rt, 1), jnp.int32),
        scratch_shapes=[pltpu.VMEM((rt, 128), jnp.float32),
                        pltpu.VMEM((rt, 128), jnp.int32)],
    )(x)


def kernel(tensor):
    return _tc_argmin(tensor)
